# baseline (speedup 1.0000x reference)
import jax
import jax.numpy as jnp
from jax import lax
from jax.experimental import pallas as pl
from jax.experimental.pallas import tpu as pltpu
from jax.experimental.pallas import tpu_sc as plsc

SEQ = 1569
SEQP = 1600
H = 16
HG = 8
NC = 2
NS = 16
L = 16
RPW = 104
G = 8
NCHUNK = RPW // G
NJ = 1600
NJV = NJ // L
PLANE = SEQ * SEQ + 7
CHW = G * SEQ


def _sc_bias_body(tab_hbm, rpi_hbm, from_hbm, to_hbm, out_hbm,
                  tab_v, to_v, fidx_v, rows_v, pidx_v, out_v, row_v,
                  sem_in, sem_out):
    g = lax.axis_index("c")
    r = lax.axis_index("s")
    pltpu.sync_copy(tab_hbm.at[g], tab_v)
    pltpu.sync_copy(to_hbm, to_v)
    n_i = jnp.minimum(RPW, SEQ - r * RPW)
    row0 = r * RPW
    h0 = g * HG
    iotav = lax.iota(jnp.int32, L)

    def chunk_body(k, carry):
        @pl.when(k * G + G <= n_i)
        def _():
            i0 = row0 + k * G
            pltpu.sync_copy(from_hbm.at[r, k], fidx_v)
            pltpu.async_copy(rpi_hbm.at[fidx_v], rows_v, sem_in).wait()
            for b in range(G):
                def permute(jv, cc):
                    tvec = to_v[pl.ds(jv * L, L)]
                    rvec = plsc.load_gather(rows_v.at[b], [tvec])
                    pidx_v[b, pl.ds(jv * L, L)] = rvec * HG
                    return cc
                lax.fori_loop(0, NJV, permute, 0, unroll=4)
            for h in range(HG):
                for b in range(G):
                    def heads(jv, cc):
                        jcol = jv * L + iotav
                        base = pidx_v[b, pl.ds(jv * L, L)]
                        val = plsc.load_gather(tab_v, [base + h])
                        plsc.store_scatter(out_v, [b * SEQ + jcol], val,
                                           mask=jcol < SEQ)
                        return cc
                    lax.fori_loop(0, NJV, heads, 0, unroll=4)
                pltpu.async_copy(
                    out_v,
                    out_hbm.at[pl.ds((h0 + h) * PLANE + i0 * SEQ, CHW)],
                    sem_out).wait()
        return carry

    lax.fori_loop(0, NCHUNK, chunk_body, 0)

    @pl.when((r == NS - 1) & (n_i % G != 0))
    def _():
        pltpu.sync_copy(from_hbm.at[NS - 1, 1], fidx_v)
        pltpu.async_copy(rpi_hbm.at[fidx_v], rows_v, sem_in).wait()

        def permute(jv, cc):
            tvec = to_v[pl.ds(jv * L, L)]
            rvec = plsc.load_gather(rows_v.at[0], [tvec])
            pidx_v[0, pl.ds(jv * L, L)] = rvec * HG
            return cc
        lax.fori_loop(0, NJV, permute, 0, unroll=4)
        for h in range(HG):
            def heads(jv, cc):
                jcol = jv * L + iotav
                base = pidx_v[0, pl.ds(jv * L, L)]
                val = plsc.load_gather(tab_v, [base + h])
                plsc.store_scatter(row_v, [jcol], val, mask=jcol < SEQ)
                return cc
            lax.fori_loop(0, NJV, heads, 0, unroll=4)
            pltpu.async_copy(
                row_v,
                out_hbm.at[pl.ds((h0 + h) * PLANE + (SEQ - 1) * SEQ, SEQ + 7)],
                sem_out).wait()


def kernel(relative_position_bias_table, relative_position_index, from_idx, to_idx):
    tab = relative_position_bias_table.astype(jnp.float32)
    nrel = tab.shape[0]
    tabf = jnp.stack([tab[:, :HG].reshape(-1), tab[:, HG:].reshape(-1)])
    rpi = jnp.pad(relative_position_index.astype(jnp.int32),
                  ((0, 0), (0, SEQP - SEQ)))
    fi = from_idx.astype(jnp.int32)
    base3d = jnp.pad(fi, (0, NS * RPW - SEQ)).reshape(NS, NCHUNK, G)
    tail_chunk = jnp.broadcast_to(fi[SEQ - 1:], (1, G))
    row15 = jnp.concatenate(
        [base3d[NS - 1, :1], tail_chunk, base3d[NS - 1, 2:]], axis=0)
    from2d = jnp.concatenate([base3d[:NS - 1], row15[None]], axis=0)
    to_pad = jnp.pad(to_idx.astype(jnp.int32), (0, NJ - SEQ))
    mesh = plsc.VectorSubcoreMesh(core_axis_name="c", subcore_axis_name="s",
                                  num_cores=NC, num_subcores=NS)
    f = pl.kernel(
        _sc_bias_body,
        out_type=jax.ShapeDtypeStruct((H * PLANE,), jnp.float32),
        mesh=mesh,
        compiler_params=pltpu.CompilerParams(use_tc_tiling_on_sc=False,
                                             needs_layout_passes=False),
        scratch_types=[
            pltpu.VMEM((nrel * HG,), jnp.float32),
            pltpu.VMEM((NJ,), jnp.int32),
            pltpu.VMEM((G,), jnp.int32),
            pltpu.VMEM((G, SEQP), jnp.int32),
            pltpu.VMEM((G, NJ), jnp.int32),
            pltpu.VMEM((CHW,), jnp.float32),
            pltpu.VMEM((SEQ + 7,), jnp.float32),
            pltpu.SemaphoreType.DMA,
            pltpu.SemaphoreType.DMA,
        ],
    )
    flat = f(tabf, rpi, from2d, to_pad)
    return flat.reshape(H, PLANE)[:, :SEQ * SEQ].reshape(H, SEQ, SEQ)


# speedup vs baseline: 4.3477x; 4.3477x over previous
import jax
import jax.numpy as jnp
from jax import lax
from jax.experimental import pallas as pl
from jax.experimental.pallas import tpu as pltpu
from jax.experimental.pallas import tpu_sc as plsc

SEQ = 1569
SEQP = 1600
H = 16
HG = 8
NC = 2
NS = 16
L = 16
RPW = 104
G = 8
GH = 4
NCHUNK = RPW // G
NJ = 1600
NJV = NJ // L
NJVF = (SEQ - 1) // L


def _sc_bias_body(tab_hbm, rpi_hbm, from_hbm, to_hbm, out_hbm,
                  tab_v, to_v, fidx_v, rows_v, pidx_v, out_a, out_b,
                  sem_in, sem_out):
    g = lax.axis_index("c")
    r = lax.axis_index("s")
    n_i = jnp.minimum(RPW, SEQ - r * RPW)
    row0 = r * RPW
    h0 = g * HG
    lane0 = lax.iota(jnp.int32, L) == 0
    out_bufs = (out_a, out_b)

    pltpu.sync_copy(from_hbm.at[r, 0], fidx_v)
    pltpu.async_copy(rpi_hbm.at[fidx_v], rows_v, sem_in)
    pltpu.sync_copy(tab_hbm.at[g], tab_v)
    pltpu.sync_copy(to_hbm, to_v)

    def chunk_body(k, carry):
        @pl.when(k * G < n_i)
        def _():
            i0 = jnp.minimum(row0 + k * G, SEQ - G)
            pltpu.make_async_copy(rpi_hbm.at[fidx_v], rows_v, sem_in).wait()
            for b in range(G):
                def permute(jv, cc):
                    tvec = to_v[pl.ds(jv * L, L)]
                    rvec = plsc.load_gather(rows_v.at[b], [tvec])
                    pidx_v[b, pl.ds(jv * L, L)] = rvec * HG
                    return cc
                lax.fori_loop(0, NJV, permute, 0, unroll=4)
            @pl.when((k + 1) * G < n_i)
            def _():
                pltpu.sync_copy(from_hbm.at[r, k + 1], fidx_v)
                pltpu.async_copy(rpi_hbm.at[fidx_v], rows_v, sem_in)
            pending = [None, None]
            for h in range(HG):
                for half in range(2):
                    pp = (2 * h + half) % 2
                    buf = out_bufs[pp]
                    if pending[pp] is not None:
                        pending[pp].wait()
                    for bb in range(GH):
                        b = half * GH + bb
                        def heads(jv, cc):
                            base = pidx_v[b, pl.ds(jv * L, L)]
                            buf[bb, pl.ds(jv * L, L)] = plsc.load_gather(
                                tab_v, [base + h])
                            return cc
                        lax.fori_loop(0, NJVF, heads, 0, unroll=4)
                        basev = pidx_v[b, pl.ds(SEQ - 1, L)]
                        valv = plsc.load_gather(tab_v, [basev + h])
                        plsc.store_scatter(
                            buf,
                            [jnp.full((L,), bb, jnp.int32),
                             jnp.full((L,), SEQ - 1, jnp.int32)],
                            valv, mask=lane0)
                    pending[pp] = pltpu.async_copy(
                        buf,
                        out_hbm.at[h0 + h, pl.ds(i0 + half * GH, GH)],
                        sem_out)
            pending[0].wait()
            pending[1].wait()
        return carry

    lax.fori_loop(0, NCHUNK, chunk_body, 0)


def kernel(relative_position_bias_table, relative_position_index, from_idx, to_idx):
    tab = relative_position_bias_table.astype(jnp.float32)
    nrel = tab.shape[0]
    tabf = jnp.stack([tab[:, :HG].reshape(-1), tab[:, HG:].reshape(-1)])
    rpi = jnp.pad(relative_position_index.astype(jnp.int32),
                  ((0, 0), (0, SEQP - SEQ)))
    fi = from_idx.astype(jnp.int32)
    base3d = jnp.pad(fi, (0, NS * RPW - SEQ)).reshape(NS, NCHUNK, G)
    row15 = jnp.concatenate(
        [base3d[NS - 1, :1], fi[SEQ - G:][None], base3d[NS - 1, 2:]], axis=0)
    from2d = jnp.concatenate([base3d[:NS - 1], row15[None]], axis=0)
    to_pad = jnp.pad(to_idx.astype(jnp.int32), (0, NJ - SEQ))
    mesh = plsc.VectorSubcoreMesh(core_axis_name="c", subcore_axis_name="s",
                                  num_cores=NC, num_subcores=NS)
    f = pl.kernel(
        _sc_bias_body,
        out_type=jax.ShapeDtypeStruct((H, SEQ, SEQ), jnp.float32),
        mesh=mesh,
        compiler_params=pltpu.CompilerParams(use_tc_tiling_on_sc=False,
                                             needs_layout_passes=False),
        scratch_types=[
            pltpu.VMEM((nrel * HG,), jnp.float32),
            pltpu.VMEM((NJ,), jnp.int32),
            pltpu.VMEM((G,), jnp.int32),
            pltpu.VMEM((G, SEQP), jnp.int32),
            pltpu.VMEM((G, NJ), jnp.int32),
            pltpu.VMEM((GH, SEQ), jnp.float32),
            pltpu.VMEM((GH, SEQ), jnp.float32),
            pltpu.SemaphoreType.DMA,
            pltpu.SemaphoreType.DMA,
        ],
    )
    return f(tabf, rpi, from2d, to_pad)


# speedup vs baseline: 5.6383x; 1.2968x over previous
import jax
import jax.numpy as jnp
from jax import lax
from jax.experimental import pallas as pl
from jax.experimental.pallas import tpu as pltpu
from jax.experimental.pallas import tpu_sc as plsc

SEQ = 1569
SEQP = 1600
H = 16
HG = 8
NC = 2
NS = 16
L = 16
RPW = 104
G = 8
GH = 4
NCHUNK = RPW // G
NJ = 1600
NJV = NJ // L
NJVF = (SEQ - 1) // L


def _sc_bias_body(tab_hbm, rpi_hbm, from_hbm, to_hbm, out_hbm,
                  tab_v, to_v, fidx_v, rows_v, pidx_v, out_a, out_b,
                  sem_in, sem_out):
    g = lax.axis_index("c")
    r = lax.axis_index("s")
    n_i = jnp.minimum(RPW, SEQ - r * RPW)
    row0 = r * RPW
    h0 = g * HG
    lane0 = lax.iota(jnp.int32, L) == 0
    out_bufs = (out_a, out_b)

    pltpu.sync_copy(from_hbm.at[r, 0], fidx_v)
    pltpu.async_copy(rpi_hbm.at[fidx_v], rows_v, sem_in)
    pltpu.sync_copy(tab_hbm.at[g], tab_v)
    pltpu.sync_copy(to_hbm, to_v)

    def chunk_body(k, carry):
        @pl.when(k * G < n_i)
        def _():
            i0 = jnp.minimum(row0 + k * G, SEQ - G)
            pltpu.make_async_copy(rpi_hbm.at[fidx_v], rows_v, sem_in).wait()
            for b in range(G):
                def permute(jv, cc):
                    tvec = to_v[pl.ds(jv * L, L)]
                    rvec = plsc.load_gather(rows_v.at[b], [tvec])
                    pidx_v[b, pl.ds(jv * L, L)] = rvec * HG
                    return cc
                lax.fori_loop(0, NJV, permute, 0)
            @pl.when((k + 1) * G < n_i)
            def _():
                pltpu.sync_copy(from_hbm.at[r, k + 1], fidx_v)
                pltpu.async_copy(rpi_hbm.at[fidx_v], rows_v, sem_in)
            pending = [None, None]
            for h in range(HG):
                for half in range(2):
                    pp = (2 * h + half) % 2
                    buf = out_bufs[pp]
                    if pending[pp] is not None:
                        pending[pp].wait()
                    for bb in range(GH):
                        b = half * GH + bb
                        def heads(jv, cc):
                            base = pidx_v[b, pl.ds(jv * L, L)]
                            buf[bb, pl.ds(jv * L, L)] = plsc.load_gather(
                                tab_v, [base + h])
                            return cc
                        lax.fori_loop(0, NJVF, heads, 0)
                        basev = pidx_v[b, pl.ds(SEQ - 1, L)]
                        valv = plsc.load_gather(tab_v, [basev + h])
                        plsc.store_scatter(
                            buf,
                            [jnp.full((L,), bb, jnp.int32),
                             jnp.full((L,), SEQ - 1, jnp.int32)],
                            valv, mask=lane0)
                    pending[pp] = pltpu.async_copy(
                        buf,
                        out_hbm.at[h0 + h, pl.ds(i0 + half * GH, GH)],
                        sem_out)
            pending[0].wait()
            pending[1].wait()
        return carry

    lax.fori_loop(0, NCHUNK, chunk_body, 0)


def kernel(relative_position_bias_table, relative_position_index, from_idx, to_idx):
    tab = relative_position_bias_table.astype(jnp.float32)
    nrel = tab.shape[0]
    tabf = jnp.stack([tab[:, :HG].reshape(-1), tab[:, HG:].reshape(-1)])
    rpi = jnp.pad(relative_position_index.astype(jnp.int32),
                  ((0, 0), (0, SEQP - SEQ)))
    fi = from_idx.astype(jnp.int32)
    base3d = jnp.pad(fi, (0, NS * RPW - SEQ)).reshape(NS, NCHUNK, G)
    row15 = jnp.concatenate(
        [base3d[NS - 1, :1], fi[SEQ - G:][None], base3d[NS - 1, 2:]], axis=0)
    from2d = jnp.concatenate([base3d[:NS - 1], row15[None]], axis=0)
    to_pad = jnp.pad(to_idx.astype(jnp.int32), (0, NJ - SEQ))
    mesh = plsc.VectorSubcoreMesh(core_axis_name="c", subcore_axis_name="s",
                                  num_cores=NC, num_subcores=NS)
    f = pl.kernel(
        _sc_bias_body,
        out_type=jax.ShapeDtypeStruct((H, SEQ, SEQ), jnp.float32),
        mesh=mesh,
        compiler_params=pltpu.CompilerParams(use_tc_tiling_on_sc=False,
                                             needs_layout_passes=False),
        scratch_types=[
            pltpu.VMEM((nrel * HG,), jnp.float32),
            pltpu.VMEM((NJ,), jnp.int32),
            pltpu.VMEM((G,), jnp.int32),
            pltpu.VMEM((G, SEQP), jnp.int32),
            pltpu.VMEM((G, NJ), jnp.int32),
            pltpu.VMEM((GH, SEQ), jnp.float32),
            pltpu.VMEM((GH, SEQ), jnp.float32),
            pltpu.SemaphoreType.DMA,
            pltpu.SemaphoreType.DMA,
        ],
    )
    return f(tabf, rpi, from2d, to_pad)


# speedup vs baseline: 6.8306x; 1.2115x over previous
import jax
import jax.numpy as jnp
from jax import lax
from jax.experimental import pallas as pl
from jax.experimental.pallas import tpu as pltpu
from jax.experimental.pallas import tpu_sc as plsc

SEQ = 1569
SEQP = 1600
H = 16
HG = 8
NC = 2
NS = 16
L = 16
RPW = 104
G = 8
GH = 4
NCHUNK = RPW // G
NJ = 1600
NJV = NJ // L
NJVF = (SEQ - 1) // L


def _sc_bias_body(tab_hbm, rpi_hbm, from_hbm, to_hbm, out_hbm,
                  tab_v, to_v, fidx_v, rows_v, pidx_v, out_a, out_b,
                  sem_in, sem_out):
    g = lax.axis_index("c")
    r = lax.axis_index("s")
    n_i = jnp.minimum(RPW, SEQ - r * RPW)
    row0 = r * RPW
    h0 = g * HG
    lane0 = lax.iota(jnp.int32, L) == 0
    out_bufs = (out_a, out_b)

    pltpu.sync_copy(from_hbm.at[r, 0], fidx_v)
    pltpu.async_copy(rpi_hbm.at[fidx_v], rows_v, sem_in)
    pltpu.sync_copy(tab_hbm.at[g], tab_v)
    pltpu.sync_copy(to_hbm, to_v)

    def chunk_body(k, carry):
        @pl.when(k * G < n_i)
        def _():
            i0 = jnp.minimum(row0 + k * G, SEQ - G)
            pltpu.make_async_copy(rpi_hbm.at[fidx_v], rows_v, sem_in).wait()
            for b in range(G):
                @plsc.parallel_loop(0, NJV, unroll=4)
                def permute(jv):
                    tvec = to_v[pl.ds(jv * L, L)]
                    rvec = plsc.load_gather(rows_v.at[b], [tvec])
                    pidx_v[b, pl.ds(jv * L, L)] = rvec * HG
            @pl.when((k + 1) * G < n_i)
            def _():
                pltpu.sync_copy(from_hbm.at[r, k + 1], fidx_v)
                pltpu.async_copy(rpi_hbm.at[fidx_v], rows_v, sem_in)
            pending = [None, None]
            for h in range(HG):
                for half in range(2):
                    pp = (2 * h + half) % 2
                    buf = out_bufs[pp]
                    if pending[pp] is not None:
                        pending[pp].wait()
                    for bb in range(GH):
                        b = half * GH + bb
                        @plsc.parallel_loop(0, NJVF, unroll=4)
                        def heads(jv):
                            base = pidx_v[b, pl.ds(jv * L, L)]
                            buf[bb, pl.ds(jv * L, L)] = plsc.load_gather(
                                tab_v, [base + h])
                        basev = pidx_v[b, pl.ds(SEQ - 1, L)]
                        valv = plsc.load_gather(tab_v, [basev + h])
                        plsc.store_scatter(
                            buf,
                            [jnp.full((L,), bb, jnp.int32),
                             jnp.full((L,), SEQ - 1, jnp.int32)],
                            valv, mask=lane0)
                    pending[pp] = pltpu.async_copy(
                        buf,
                        out_hbm.at[h0 + h, pl.ds(i0 + half * GH, GH)],
                        sem_out)
            pending[0].wait()
            pending[1].wait()
        return carry

    lax.fori_loop(0, NCHUNK, chunk_body, 0)


def kernel(relative_position_bias_table, relative_position_index, from_idx, to_idx):
    tab = relative_position_bias_table.astype(jnp.float32)
    nrel = tab.shape[0]
    tabf = jnp.stack([tab[:, :HG].reshape(-1), tab[:, HG:].reshape(-1)])
    rpi = jnp.pad(relative_position_index.astype(jnp.int32),
                  ((0, 0), (0, SEQP - SEQ)))
    fi = from_idx.astype(jnp.int32)
    base3d = jnp.pad(fi, (0, NS * RPW - SEQ)).reshape(NS, NCHUNK, G)
    row15 = jnp.concatenate(
        [base3d[NS - 1, :1], fi[SEQ - G:][None], base3d[NS - 1, 2:]], axis=0)
    from2d = jnp.concatenate([base3d[:NS - 1], row15[None]], axis=0)
    to_pad = jnp.pad(to_idx.astype(jnp.int32), (0, NJ - SEQ))
    mesh = plsc.VectorSubcoreMesh(core_axis_name="c", subcore_axis_name="s",
                                  num_cores=NC, num_subcores=NS)
    f = pl.kernel(
        _sc_bias_body,
        out_type=jax.ShapeDtypeStruct((H, SEQ, SEQ), jnp.float32),
        mesh=mesh,
        compiler_params=pltpu.CompilerParams(use_tc_tiling_on_sc=False,
                                             needs_layout_passes=False),
        scratch_types=[
            pltpu.VMEM((nrel * HG,), jnp.float32),
            pltpu.VMEM((NJ,), jnp.int32),
            pltpu.VMEM((G,), jnp.int32),
            pltpu.VMEM((G, SEQP), jnp.int32),
            pltpu.VMEM((G, NJ), jnp.int32),
            pltpu.VMEM((GH, SEQ), jnp.float32),
            pltpu.VMEM((GH, SEQ), jnp.float32),
            pltpu.SemaphoreType.DMA,
            pltpu.SemaphoreType.DMA,
        ],
    )
    return f(tabf, rpi, from2d, to_pad)


# speedup vs baseline: 17.1477x; 2.5104x over previous
import jax
import jax.numpy as jnp
from jax import lax
from jax.experimental import pallas as pl
from jax.experimental.pallas import tpu as pltpu
from jax.experimental.pallas import tpu_sc as plsc

SEQ = 1569
SEQP = 1600
H = 16
HG = 8
NC = 2
NS = 16
L = 16
RPW = 104
G = 8
NCHUNK = RPW // G
NBAND = (SEQ + G - 1) // G
NCT = (SEQ + 127) // 128
NJ = NCT * 128
NJV = NJ // L
BAND = NCT * G * 128


def _sc_bias_body(tab_hbm, rpi_hbm, from_hbm, to_hbm, out_hbm,
                  tab_v, to_v, fidx_v, rows_v, pidx_v, out_v,
                  sem_in, sem_out):
    g = lax.axis_index("c")
    r = lax.axis_index("s")
    n_i = jnp.minimum(RPW, SEQ - r * RPW)
    h0 = g * HG

    pltpu.sync_copy(from_hbm.at[r, 0], fidx_v)
    pltpu.async_copy(rpi_hbm.at[fidx_v], rows_v, sem_in)
    pltpu.sync_copy(tab_hbm.at[g], tab_v)
    pltpu.sync_copy(to_hbm, to_v)

    def chunk_body(k, carry):
        @pl.when(k * G < n_i)
        def _():
            band = r * NCHUNK + k
            pltpu.make_async_copy(rpi_hbm.at[fidx_v], rows_v, sem_in).wait()
            for b in range(G):
                @plsc.parallel_loop(0, NJV, unroll=4)
                def permute(jv):
                    tvec = to_v[pl.ds(jv * L, L)]
                    rvec = plsc.load_gather(rows_v.at[b], [tvec])
                    pidx_v[b, pl.ds(jv * L, L)] = rvec * HG
            @pl.when((k + 1) * G < n_i)
            def _():
                pltpu.sync_copy(from_hbm.at[r, k + 1], fidx_v)
                pltpu.async_copy(rpi_hbm.at[fidx_v], rows_v, sem_in)
            for h in range(HG):
                for b in range(G):
                    @plsc.parallel_loop(0, NJV, unroll=4)
                    def heads(jv):
                        base = pidx_v[b, pl.ds(jv * L, L)]
                        off = (jv // G) * 1024 + b * 128 + (jv % G) * L
                        out_v[pl.ds(off, L)] = plsc.load_gather(
                            tab_v, [base + h])
                pltpu.async_copy(
                    out_v,
                    out_hbm.at[pl.ds(((h0 + h) * NBAND + band) * BAND, BAND)],
                    sem_out).wait()
        return carry

    lax.fori_loop(0, NCHUNK, chunk_body, 0)


def kernel(relative_position_bias_table, relative_position_index, from_idx, to_idx):
    tab = relative_position_bias_table.astype(jnp.float32)
    nrel = tab.shape[0]
    tabf = jnp.stack([tab[:, :HG].reshape(-1), tab[:, HG:].reshape(-1)])
    rpi = jnp.pad(relative_position_index.astype(jnp.int32),
                  ((0, 0), (0, SEQP - SEQ)))
    fi = from_idx.astype(jnp.int32)
    base3d = jnp.pad(fi, (0, NS * RPW - SEQ)).reshape(NS, NCHUNK, G)
    tail_chunk = jnp.broadcast_to(fi[SEQ - 1:], (1, G))
    row15 = jnp.concatenate(
        [base3d[NS - 1, :1], tail_chunk, base3d[NS - 1, 2:]], axis=0)
    from2d = jnp.concatenate([base3d[:NS - 1], row15[None]], axis=0)
    to_pad = jnp.pad(to_idx.astype(jnp.int32), (0, NJ - SEQ))
    mesh = plsc.VectorSubcoreMesh(core_axis_name="c", subcore_axis_name="s",
                                  num_cores=NC, num_subcores=NS)
    f = pl.kernel(
        _sc_bias_body,
        out_type=jax.ShapeDtypeStruct((H * NBAND * BAND,), jnp.float32),
        mesh=mesh,
        compiler_params=pltpu.CompilerParams(use_tc_tiling_on_sc=False,
                                             needs_layout_passes=False),
        scratch_types=[
            pltpu.VMEM((nrel * HG,), jnp.float32),
            pltpu.VMEM((NJ,), jnp.int32),
            pltpu.VMEM((G,), jnp.int32),
            pltpu.VMEM((G, SEQP), jnp.int32),
            pltpu.VMEM((G, NJ), jnp.int32),
            pltpu.VMEM((BAND,), jnp.float32),
            pltpu.SemaphoreType.DMA,
            pltpu.SemaphoreType.DMA,
        ],
    )
    flat = f(tabf, rpi, from2d, to_pad)
    out5 = flat.reshape(H, NBAND, NCT, G, 128)
    return (out5.transpose(0, 1, 3, 2, 4)
            .reshape(H, NBAND * G, NCT * 128)[:, :SEQ, :SEQ])
